# baseline (speedup 1.0000x reference)
import functools

import jax
import jax.numpy as jnp
from jax import lax
from jax.experimental import pallas as pl
from jax.experimental.pallas import tpu as pltpu
from jax.experimental.pallas import tpu_sc as plsc

_WIN = 128
_NBUF = 6


def _make_sc_gather(n, d, win):
    mesh = plsc.VectorSubcoreMesh(
        core_axis_name="core", subcore_axis_name="subcore"
    )

    @functools.partial(
        pl.kernel,
        mesh=mesh,
        out_type=jax.ShapeDtypeStruct((n, d), jnp.float32),
        compiler_params=pltpu.CompilerParams(use_tc_tiling_on_sc=False),
    )
    def gather_kernel(tab_hbm, idx_hbm, out_hbm):
        def body(i_vmem, o_vmem):
            pltpu.sync_copy(tab_hbm.at[i_vmem.at[0, 0]], o_vmem)

        pltpu.emit_pipeline(
            body,
            grid=(n // win,),
            in_specs=[
                pl.BlockSpec((1, 1, win), index_map=lambda i: (i, 0, 0))
            ],
            out_specs=[pl.BlockSpec((win, d), index_map=lambda i: (i, 0))],
            core_axis_name=("core", "subcore"),
            dimension_semantics=(pltpu.PARALLEL,),
        )(idx_hbm, out_hbm)

    return gather_kernel


def _make_proj_body(l_off):
    def _proj_body(w_ref, x_ref, b_ref, o_hbm, buf, sem):
        l = pl.program_id(0)
        nl = pl.num_programs(0)
        jm = lax.rem(l, _NBUF)

        for j in range(_NBUF):

            @pl.when(jnp.logical_and(l >= _NBUF, jm == j))
            def _(j=j):
                pltpu.make_async_copy(
                    buf.at[j], o_hbm.at[l_off + l - _NBUF], sem.at[j]
                ).wait()

        res = lax.dot_general(
            w_ref[...],
            x_ref[l],
            (((1,), (1,)), ((), ())),
            preferred_element_type=jnp.float32,
        )
        dst = buf.at[jm]
        dst[...] = res + b_ref[...]
        for j in range(_NBUF):

            @pl.when(jm == j)
            def _(j=j):
                pltpu.make_async_copy(
                    buf.at[j], o_hbm.at[l_off + l], sem.at[j]
                ).start()

        @pl.when(l == nl - 1)
        def _():
            for j in range(_NBUF):
                pltpu.make_async_copy(
                    buf.at[j], o_hbm.at[l_off + l], sem.at[j]
                ).wait()

    return _proj_body


def _proj_body_aliased(l_off):
    inner = _make_proj_body(l_off)

    def body(w_ref, x_ref, b_ref, prev_ref, o_hbm, buf, sem):
        del prev_ref
        inner(w_ref, x_ref, b_ref, o_hbm, buf, sem)

    return body


def kernel(input_ids, emb_table, W, b):
    B, L = input_ids.shape
    V, E = emb_table.shape
    L1 = L // 2

    ids_t = input_ids.T.astype(jnp.int32)

    def sc_gather_chunk(l0, l1):
        nc = (l1 - l0) * B
        idx = ids_t[l0:l1].reshape(nc // _WIN, 1, _WIN)
        x = _make_sc_gather(nc, E, _WIN)(emb_table, idx)
        return x.reshape(l1 - l0, B, E).astype(jnp.bfloat16)

    xa = sc_gather_chunk(0, L1)
    xb = sc_gather_chunk(L1, L)

    wb = W.astype(jnp.bfloat16)
    b2 = b.reshape(V, 1)
    out_shape = jax.ShapeDtypeStruct((L, V, B), jnp.float32)
    scratch = [
        pltpu.VMEM((_NBUF, V, B), jnp.float32),
        pltpu.SemaphoreType.DMA((_NBUF,)),
    ]

    out_a = pl.pallas_call(
        _make_proj_body(0),
        grid=(L1,),
        in_specs=[
            pl.BlockSpec((V, E), lambda i: (0, 0)),
            pl.BlockSpec((L1, B, E), lambda i: (0, 0, 0)),
            pl.BlockSpec((V, 1), lambda i: (0, 0)),
        ],
        out_specs=pl.BlockSpec(memory_space=pl.ANY),
        out_shape=out_shape,
        scratch_shapes=scratch,
    )(wb, xa, b2)

    out_t = pl.pallas_call(
        _proj_body_aliased(L1),
        grid=(L - L1,),
        in_specs=[
            pl.BlockSpec((V, E), lambda i: (0, 0)),
            pl.BlockSpec((L - L1, B, E), lambda i: (0, 0, 0)),
            pl.BlockSpec((V, 1), lambda i: (0, 0)),
            pl.BlockSpec(memory_space=pl.ANY),
        ],
        out_specs=pl.BlockSpec(memory_space=pl.ANY),
        out_shape=out_shape,
        input_output_aliases={3: 0},
        scratch_shapes=scratch,
    )(wb, xb, b2, out_a)

    return out_t.transpose(2, 0, 1)


# speedup vs baseline: 1.0032x; 1.0032x over previous
import functools

import jax
import jax.numpy as jnp
from jax import lax
from jax.experimental import pallas as pl
from jax.experimental.pallas import tpu as pltpu
from jax.experimental.pallas import tpu_sc as plsc

_WIN = 128
_NBUF = 4


def _make_sc_gather(n, d, win):
    mesh = plsc.VectorSubcoreMesh(
        core_axis_name="core", subcore_axis_name="subcore"
    )

    @functools.partial(
        pl.kernel,
        mesh=mesh,
        out_type=jax.ShapeDtypeStruct((n, d), jnp.float32),
        compiler_params=pltpu.CompilerParams(use_tc_tiling_on_sc=False),
    )
    def gather_kernel(tab_hbm, idx_hbm, out_hbm):
        def body(i_vmem, o_vmem):
            pltpu.sync_copy(tab_hbm.at[i_vmem.at[0, 0]], o_vmem)

        pltpu.emit_pipeline(
            body,
            grid=(n // win,),
            in_specs=[
                pl.BlockSpec((1, 1, win), index_map=lambda i: (i, 0, 0))
            ],
            out_specs=[pl.BlockSpec((win, d), index_map=lambda i: (i, 0))],
            core_axis_name=("core", "subcore"),
            dimension_semantics=(pltpu.PARALLEL,),
        )(idx_hbm, out_hbm)

    return gather_kernel


def _make_proj_body(l_off):
    def _proj_body(w_ref, x_ref, b_ref, o_hbm, buf, sem):
        l = pl.program_id(0)
        nl = pl.num_programs(0)
        jm = lax.rem(l, _NBUF)

        for j in range(_NBUF):

            @pl.when(jnp.logical_and(l >= _NBUF, jm == j))
            def _(j=j):
                pltpu.make_async_copy(
                    buf.at[j], o_hbm.at[l_off + l - _NBUF], sem.at[j]
                ).wait()

        res = lax.dot_general(
            w_ref[...],
            x_ref[l],
            (((1,), (1,)), ((), ())),
            preferred_element_type=jnp.float32,
        )
        dst = buf.at[jm]
        dst[...] = res + b_ref[...]
        for j in range(_NBUF):

            @pl.when(jm == j)
            def _(j=j):
                pltpu.make_async_copy(
                    buf.at[j], o_hbm.at[l_off + l], sem.at[j]
                ).start()

        @pl.when(l == nl - 1)
        def _():
            for j in range(_NBUF):
                pltpu.make_async_copy(
                    buf.at[j], o_hbm.at[l_off + l], sem.at[j]
                ).wait()

    return _proj_body


def _proj_body_aliased(l_off):
    inner = _make_proj_body(l_off)

    def body(w_ref, x_ref, b_ref, prev_ref, o_hbm, buf, sem):
        del prev_ref
        inner(w_ref, x_ref, b_ref, o_hbm, buf, sem)

    return body


def kernel(input_ids, emb_table, W, b):
    B, L = input_ids.shape
    V, E = emb_table.shape
    L1 = 2 * L // 5

    ids_t = input_ids.T.astype(jnp.int32)

    def sc_gather_chunk(l0, l1):
        nc = (l1 - l0) * B
        idx = ids_t[l0:l1].reshape(nc // _WIN, 1, _WIN)
        x = _make_sc_gather(nc, E, _WIN)(emb_table, idx)
        return x.reshape(l1 - l0, B, E).astype(jnp.bfloat16)

    xa = sc_gather_chunk(0, L1)
    xb = sc_gather_chunk(L1, L)

    wb = W.astype(jnp.bfloat16)
    b2 = b.reshape(V, 1)
    out_shape = jax.ShapeDtypeStruct((L, V, B), jnp.float32)
    scratch = [
        pltpu.VMEM((_NBUF, V, B), jnp.float32),
        pltpu.SemaphoreType.DMA((_NBUF,)),
    ]

    out_a = pl.pallas_call(
        _make_proj_body(0),
        grid=(L1,),
        in_specs=[
            pl.BlockSpec((V, E), lambda i: (0, 0)),
            pl.BlockSpec((L1, B, E), lambda i: (0, 0, 0)),
            pl.BlockSpec((V, 1), lambda i: (0, 0)),
        ],
        out_specs=pl.BlockSpec(memory_space=pl.ANY),
        out_shape=out_shape,
        scratch_shapes=scratch,
    )(wb, xa, b2)

    out_t = pl.pallas_call(
        _proj_body_aliased(L1),
        grid=(L - L1,),
        in_specs=[
            pl.BlockSpec((V, E), lambda i: (0, 0)),
            pl.BlockSpec((L - L1, B, E), lambda i: (0, 0, 0)),
            pl.BlockSpec((V, 1), lambda i: (0, 0)),
            pl.BlockSpec(memory_space=pl.ANY),
        ],
        out_specs=pl.BlockSpec(memory_space=pl.ANY),
        out_shape=out_shape,
        input_output_aliases={3: 0},
        scratch_shapes=scratch,
    )(wb, xb, b2, out_a)

    return out_t.transpose(2, 0, 1)


# speedup vs baseline: 1.0070x; 1.0038x over previous
import functools

import jax
import jax.numpy as jnp
from jax import lax
from jax.experimental import pallas as pl
from jax.experimental.pallas import tpu as pltpu
from jax.experimental.pallas import tpu_sc as plsc

_WIN = 128
_NBUF = 4


def _make_sc_gather(n, d, win):
    mesh = plsc.VectorSubcoreMesh(
        core_axis_name="core", subcore_axis_name="subcore"
    )

    @functools.partial(
        pl.kernel,
        mesh=mesh,
        out_type=jax.ShapeDtypeStruct((n, d), jnp.float32),
        compiler_params=pltpu.CompilerParams(use_tc_tiling_on_sc=False),
    )
    def gather_kernel(tab_hbm, idx_hbm, out_hbm):
        def body(i_vmem, o_vmem):
            pltpu.sync_copy(tab_hbm.at[i_vmem.at[0, 0]], o_vmem)

        pltpu.emit_pipeline(
            body,
            grid=(n // win,),
            in_specs=[
                pl.BlockSpec((1, 1, win), index_map=lambda i: (i, 0, 0))
            ],
            out_specs=[pl.BlockSpec((win, d), index_map=lambda i: (i, 0))],
            core_axis_name=("core", "subcore"),
            dimension_semantics=(pltpu.PARALLEL,),
        )(idx_hbm, out_hbm)

    return gather_kernel


def _make_proj_body(l_off):
    def _proj_body(w_ref, x_ref, b_ref, o_hbm, buf, sem):
        l = pl.program_id(0)
        nl = pl.num_programs(0)
        jm = lax.rem(l, _NBUF)

        for j in range(_NBUF):

            @pl.when(jnp.logical_and(l >= _NBUF, jm == j))
            def _(j=j):
                pltpu.make_async_copy(
                    buf.at[j], o_hbm.at[l_off + l - _NBUF], sem.at[j]
                ).wait()

        res = lax.dot_general(
            w_ref[...],
            x_ref[l],
            (((1,), (1,)), ((), ())),
            preferred_element_type=jnp.float32,
        )
        dst = buf.at[jm]
        dst[...] = res + b_ref[...]
        for j in range(_NBUF):

            @pl.when(jm == j)
            def _(j=j):
                pltpu.make_async_copy(
                    buf.at[j], o_hbm.at[l_off + l], sem.at[j]
                ).start()

        @pl.when(l == nl - 1)
        def _():
            for j in range(_NBUF):
                pltpu.make_async_copy(
                    buf.at[j], o_hbm.at[l_off + l], sem.at[j]
                ).wait()

    return _proj_body


def _proj_body_aliased(l_off):
    inner = _make_proj_body(l_off)

    def body(w_ref, x_ref, b_ref, prev_ref, o_hbm, buf, sem):
        del prev_ref
        inner(w_ref, x_ref, b_ref, o_hbm, buf, sem)

    return body


def kernel(input_ids, emb_table, W, b):
    B, L = input_ids.shape
    V, E = emb_table.shape
    L1 = L // 2

    ids_t = input_ids.T.astype(jnp.int32)

    def sc_gather_chunk(l0, l1):
        nc = (l1 - l0) * B
        idx = ids_t[l0:l1].reshape(nc // _WIN, 1, _WIN)
        x = _make_sc_gather(nc, E, _WIN)(emb_table, idx)
        return x.reshape(l1 - l0, B, E).astype(jnp.bfloat16)

    xa = sc_gather_chunk(0, L1)
    xb = sc_gather_chunk(L1, L)

    wb = W.astype(jnp.bfloat16)
    b2 = b.reshape(V, 1)
    out_shape = jax.ShapeDtypeStruct((L, V, B), jnp.float32)
    scratch = [
        pltpu.VMEM((_NBUF, V, B), jnp.float32),
        pltpu.SemaphoreType.DMA((_NBUF,)),
    ]

    out_a = pl.pallas_call(
        _make_proj_body(0),
        grid=(L1,),
        in_specs=[
            pl.BlockSpec((V, E), lambda i: (0, 0)),
            pl.BlockSpec((L1, B, E), lambda i: (0, 0, 0)),
            pl.BlockSpec((V, 1), lambda i: (0, 0)),
        ],
        out_specs=pl.BlockSpec(memory_space=pl.ANY),
        out_shape=out_shape,
        scratch_shapes=scratch,
    )(wb, xa, b2)

    out_t = pl.pallas_call(
        _proj_body_aliased(L1),
        grid=(L - L1,),
        in_specs=[
            pl.BlockSpec((V, E), lambda i: (0, 0)),
            pl.BlockSpec((L - L1, B, E), lambda i: (0, 0, 0)),
            pl.BlockSpec((V, 1), lambda i: (0, 0)),
            pl.BlockSpec(memory_space=pl.ANY),
        ],
        out_specs=pl.BlockSpec(memory_space=pl.ANY),
        out_shape=out_shape,
        input_output_aliases={3: 0},
        scratch_shapes=scratch,
    )(wb, xb, b2, out_a)

    return out_t.transpose(2, 0, 1)
